# baseline (speedup 1.0000x reference)
import functools

import jax
import jax.numpy as jnp
from jax import lax
from jax.experimental import pallas as pl
from jax.experimental.pallas import tpu as pltpu
from jax.experimental.pallas import tpu_sc as plsc

_G = 2
_TILE_A = 2048
_TILE = 2048
_CHUNK = 128


def _proj_body(x_ref, w_ref, b_ref, lt_ref, idx0_ref, idx1_ref, *, num_vars):
    lt = lax.dot_general(
        w_ref[...], x_ref[...], (((1,), (1,)), ((), ())),
        preferred_element_type=jnp.float32)
    lt = lt + jnp.transpose(jnp.reshape(b_ref[...], (1, 2 * num_vars)), (1, 0))
    lt_ref[...] = lt.astype(jnp.bfloat16)

    iota = lax.broadcasted_iota(jnp.int32, (num_vars, _TILE_A), 0)
    for g, idx_ref in ((0, idx0_ref), (1, idx1_ref)):
        lg = lt[g * num_vars:(g + 1) * num_vars, :]
        m = jnp.max(lg, axis=0, keepdims=True)
        k = jnp.min(jnp.where(lg == m, iota, num_vars), axis=0, keepdims=True)
        idx_ref[...] = jnp.reshape(k + g * num_vars,
                                   (_TILE_A // _CHUNK, _CHUNK))


def _project_and_select(x, W, b2, num_vars):
    rows, fsz = x.shape
    gv = W.shape[0]
    grid = rows // _TILE_A
    nchunks = rows // _CHUNK
    body = functools.partial(_proj_body, num_vars=num_vars)
    return pl.pallas_call(
        body,
        grid=(grid,),
        in_specs=[
            pl.BlockSpec((_TILE_A, fsz), lambda i: (i, 0)),
            pl.BlockSpec((gv, fsz), lambda i: (0, 0)),
            pl.BlockSpec((gv,), lambda i: (0,)),
        ],
        out_specs=[
            pl.BlockSpec((gv, _TILE_A), lambda i: (0, i)),
            pl.BlockSpec((_TILE_A // _CHUNK, _CHUNK), lambda i: (i, 0)),
            pl.BlockSpec((_TILE_A // _CHUNK, _CHUNK), lambda i: (i, 0)),
        ],
        out_shape=[
            jax.ShapeDtypeStruct((gv, rows), jnp.bfloat16),
            jax.ShapeDtypeStruct((nchunks, _CHUNK), jnp.int32),
            jax.ShapeDtypeStruct((nchunks, _CHUNK), jnp.int32),
        ],
        compiler_params=pltpu.CompilerParams(
            dimension_semantics=("parallel",)),
    )(x, W, b2)


def _stats_body(lt_ref, idx0_ref, idx1_ref, cperp_ref, pperp_ref,
                psum_acc, cnt_acc, *, num_vars, rows):
    i = pl.program_id(0)
    nsteps = pl.num_programs(0)

    @pl.when(i == 0)
    def _init():
        psum_acc[...] = jnp.zeros_like(psum_acc)
        cnt_acc[...] = jnp.zeros_like(cnt_acc)

    lt = lt_ref[...].astype(jnp.float32)
    iota = lax.broadcasted_iota(jnp.int32, (num_vars, _TILE), 0)
    ones_row = jnp.ones((1, _TILE), jnp.float32)
    for g, idx_ref in ((0, idx0_ref), (1, idx1_ref)):
        lg = lt[g * num_vars:(g + 1) * num_vars, :]
        m = jnp.max(lg, axis=0, keepdims=True)
        e = jnp.exp(lg - m)
        w_s = 1.0 / jnp.sum(e, axis=0, keepdims=True)
        psum_acc[:, g:g + 1] += lax.dot_general(
            e, w_s, (((1,), (1,)), ((), ())),
            preferred_element_type=jnp.float32)
        k = jnp.reshape(idx_ref[...], (1, _TILE)) - g * num_vars
        oh = (iota == k).astype(jnp.float32)
        cnt_acc[:, g:g + 1] += lax.dot_general(
            oh, ones_row, (((1,), (1,)), ((), ())),
            preferred_element_type=jnp.float32)

    @pl.when(i == nsteps - 1)
    def _fini():
        inv_n = 1.0 / rows
        hp = cnt_acc[...] * inv_n
        ent_h = jnp.sum(hp * jnp.log(hp + 1e-7), axis=0, keepdims=True)
        cperp_ref[...] = jnp.sum(jnp.exp(-ent_h), axis=1, keepdims=True)
        ap = psum_acc[...] * inv_n
        ent_a = jnp.sum(ap * jnp.log(ap + 1e-7), axis=0, keepdims=True)
        pperp_ref[...] = jnp.sum(jnp.exp(-ent_a), axis=1, keepdims=True)


def _perplexities(lt, idx0, idx1, num_vars):
    gv, rows = lt.shape
    grid = rows // _TILE
    body = functools.partial(_stats_body, num_vars=num_vars, rows=float(rows))
    return pl.pallas_call(
        body,
        grid=(grid,),
        in_specs=[
            pl.BlockSpec((gv, _TILE), lambda i: (0, i)),
            pl.BlockSpec((_TILE // _CHUNK, _CHUNK), lambda i: (i, 0)),
            pl.BlockSpec((_TILE // _CHUNK, _CHUNK), lambda i: (i, 0)),
        ],
        out_specs=[
            pl.BlockSpec((1, 1), lambda i: (0, 0)),
            pl.BlockSpec((1, 1), lambda i: (0, 0)),
        ],
        out_shape=[
            jax.ShapeDtypeStruct((1, 1), jnp.float32),
            jax.ShapeDtypeStruct((1, 1), jnp.float32),
        ],
        scratch_shapes=[
            pltpu.VMEM((num_vars, _G), jnp.float32),
            pltpu.VMEM((num_vars, _G), jnp.float32),
        ],
    )(lt, idx0, idx1)


def _gather_body(table_hbm, idx0_hbm, idx1_hbm, out_hbm, idx_v, rows_v, sem,
                 *, num_cores, tok_w, var_dim):
    wid = lax.axis_index("s") * num_cores + lax.axis_index("c")
    cw = tok_w // _CHUNK
    pltpu.sync_copy(idx0_hbm.at[pl.ds(wid * cw, cw)], idx_v.at[pl.ds(0, cw)])
    pltpu.sync_copy(idx1_hbm.at[pl.ds(wid * cw, cw)], idx_v.at[pl.ds(cw, cw)])
    cps = [
        pltpu.async_copy(table_hbm.at[idx_v.at[g * cw + c]],
                         rows_v.at[pl.ds((g * cw + c) * _CHUNK, _CHUNK)], sem)
        for g in range(_G) for c in range(cw)
    ]
    for c in cps:
        c.wait()
    for g in range(_G):
        pltpu.sync_copy(
            rows_v.at[pl.ds(g * tok_w, tok_w)],
            out_hbm.at[pl.ds(wid * tok_w, tok_w),
                       pl.ds(g * var_dim, var_dim)])


def _sc_gather(table, idx0, idx1, n_tok):
    var_dim = table.shape[-1]
    info = plsc.get_sparse_core_info()
    nw = info.num_cores * info.num_subcores
    tok_w = n_tok // nw
    mesh = plsc.VectorSubcoreMesh(core_axis_name="c", subcore_axis_name="s")
    body = functools.partial(_gather_body, num_cores=info.num_cores,
                             tok_w=tok_w, var_dim=var_dim)
    f = pl.kernel(
        body,
        out_type=jax.ShapeDtypeStruct((n_tok, _G * var_dim), jnp.float32),
        mesh=mesh,
        scratch_types=[
            pltpu.VMEM((_G * tok_w // _CHUNK, _CHUNK), jnp.int32),
            pltpu.VMEM((_G * tok_w, var_dim), jnp.float32),
            pltpu.SemaphoreType.DMA,
        ],
    )
    return f(table, idx0, idx1)


def kernel(x, codebook, W, b):
    bsz, tsz, fsz = x.shape
    rows = bsz * tsz
    gv = W.shape[0]
    num_vars = gv // _G
    var_dim = codebook.shape[-1]

    lt, idx0, idx1 = _project_and_select(
        x.reshape(rows, fsz), W, b, num_vars)
    q2d = _sc_gather(codebook.reshape(gv, var_dim), idx0, idx1, rows)
    cperp, pperp = _perplexities(lt, idx0, idx1, num_vars)
    q = q2d.reshape(bsz, tsz, _G * var_dim)
    return (q, gv, cperp.reshape(()), pperp.reshape(()))


# speedup vs baseline: 1.0071x; 1.0071x over previous
import functools

import jax
import jax.numpy as jnp
from jax import lax
from jax.experimental import pallas as pl
from jax.experimental.pallas import tpu as pltpu
from jax.experimental.pallas import tpu_sc as plsc

_G = 2
_TILE_A = 2048
_TILE = 2048
_CHUNK = 128


def _proj_body(x_ref, w_ref, b_ref, lt_ref, idx0_ref, idx1_ref, *, num_vars):
    lt = lax.dot_general(
        w_ref[...], x_ref[...], (((1,), (1,)), ((), ())),
        preferred_element_type=jnp.float32)
    lt = lt + jnp.transpose(jnp.reshape(b_ref[...], (1, 2 * num_vars)), (1, 0))
    lt_ref[...] = lt.astype(jnp.bfloat16)

    iota = lax.broadcasted_iota(jnp.int32, (num_vars, _TILE_A), 0)
    for g, idx_ref in ((0, idx0_ref), (1, idx1_ref)):
        lg = lt[g * num_vars:(g + 1) * num_vars, :]
        m = jnp.max(lg, axis=0, keepdims=True)
        k = jnp.min(jnp.where(lg == m, iota, num_vars), axis=0, keepdims=True)
        idx_ref[...] = jnp.reshape(k + g * num_vars,
                                   (_TILE_A // _CHUNK, _CHUNK))


def _project_and_select(x, W, b, num_vars):
    rows, fsz = x.shape
    gv = W.shape[0]
    grid = rows // _TILE_A
    nchunks = rows // _CHUNK
    body = functools.partial(_proj_body, num_vars=num_vars)
    return pl.pallas_call(
        body,
        grid=(grid,),
        in_specs=[
            pl.BlockSpec((_TILE_A, fsz), lambda i: (i, 0)),
            pl.BlockSpec((gv, fsz), lambda i: (0, 0)),
            pl.BlockSpec((gv,), lambda i: (0,)),
        ],
        out_specs=[
            pl.BlockSpec((gv, _TILE_A), lambda i: (0, i)),
            pl.BlockSpec((_TILE_A // _CHUNK, _CHUNK), lambda i: (i, 0)),
            pl.BlockSpec((_TILE_A // _CHUNK, _CHUNK), lambda i: (i, 0)),
        ],
        out_shape=[
            jax.ShapeDtypeStruct((gv, rows), jnp.bfloat16),
            jax.ShapeDtypeStruct((nchunks, _CHUNK), jnp.int32),
            jax.ShapeDtypeStruct((nchunks, _CHUNK), jnp.int32),
        ],
        compiler_params=pltpu.CompilerParams(
            dimension_semantics=("parallel",)),
    )(x, W, b)


def _stats_body(lt_ref, idx0_ref, idx1_ref, cperp_ref, pperp_ref,
                psum_acc, cnt_acc, *, num_vars, rows):
    i = pl.program_id(0)
    nsteps = pl.num_programs(0)

    @pl.when(i == 0)
    def _init():
        psum_acc[...] = jnp.zeros_like(psum_acc)
        cnt_acc[...] = jnp.zeros_like(cnt_acc)

    lt = lt_ref[...].astype(jnp.float32)
    iota = lax.broadcasted_iota(jnp.int32, (num_vars, _TILE), 0)
    ones_row = jnp.ones((1, _TILE), jnp.float32)
    for g, idx_ref in ((0, idx0_ref), (1, idx1_ref)):
        lg = lt[g * num_vars:(g + 1) * num_vars, :]
        m = jnp.max(lg, axis=0, keepdims=True)
        e = jnp.exp(lg - m)
        w_s = 1.0 / jnp.sum(e, axis=0, keepdims=True)
        psum_acc[:, g:g + 1] += lax.dot_general(
            e, w_s, (((1,), (1,)), ((), ())),
            preferred_element_type=jnp.float32)
        k = jnp.reshape(idx_ref[...], (1, _TILE)) - g * num_vars
        oh = (iota == k).astype(jnp.float32)
        cnt_acc[:, g:g + 1] += lax.dot_general(
            oh, ones_row, (((1,), (1,)), ((), ())),
            preferred_element_type=jnp.float32)

    @pl.when(i == nsteps - 1)
    def _fini():
        inv_n = 1.0 / rows
        hp = cnt_acc[...] * inv_n
        ent_h = jnp.sum(hp * jnp.log(hp + 1e-7), axis=0, keepdims=True)
        cperp_ref[...] = jnp.sum(jnp.exp(-ent_h), axis=1, keepdims=True)
        ap = psum_acc[...] * inv_n
        ent_a = jnp.sum(ap * jnp.log(ap + 1e-7), axis=0, keepdims=True)
        pperp_ref[...] = jnp.sum(jnp.exp(-ent_a), axis=1, keepdims=True)


def _perplexities(lt, idx0, idx1, num_vars):
    gv, rows = lt.shape
    grid = rows // _TILE
    body = functools.partial(_stats_body, num_vars=num_vars, rows=float(rows))
    return pl.pallas_call(
        body,
        grid=(grid,),
        in_specs=[
            pl.BlockSpec((gv, _TILE), lambda i: (0, i)),
            pl.BlockSpec((_TILE // _CHUNK, _CHUNK), lambda i: (i, 0)),
            pl.BlockSpec((_TILE // _CHUNK, _CHUNK), lambda i: (i, 0)),
        ],
        out_specs=[
            pl.BlockSpec((1, 1), lambda i: (0, 0)),
            pl.BlockSpec((1, 1), lambda i: (0, 0)),
        ],
        out_shape=[
            jax.ShapeDtypeStruct((1, 1), jnp.float32),
            jax.ShapeDtypeStruct((1, 1), jnp.float32),
        ],
        scratch_shapes=[
            pltpu.VMEM((num_vars, _G), jnp.float32),
            pltpu.VMEM((num_vars, _G), jnp.float32),
        ],
    )(lt, idx0, idx1)


def _gather_body(table_hbm, idx0_hbm, idx1_hbm, out_hbm, idx_v, rows_v, sem,
                 *, num_cores, tok_w, var_dim):
    wid = lax.axis_index("s") * num_cores + lax.axis_index("c")
    cw = tok_w // _CHUNK
    pltpu.sync_copy(idx0_hbm.at[pl.ds(wid * cw, cw)], idx_v.at[pl.ds(0, cw)])
    pltpu.sync_copy(idx1_hbm.at[pl.ds(wid * cw, cw)], idx_v.at[pl.ds(cw, cw)])
    cps = [
        pltpu.async_copy(table_hbm.at[idx_v.at[g * cw + c]],
                         rows_v.at[pl.ds((g * cw + c) * _CHUNK, _CHUNK)], sem)
        for g in range(_G) for c in range(cw)
    ]
    for c in cps:
        c.wait()
    for g in range(_G):
        pltpu.sync_copy(
            rows_v.at[pl.ds(g * tok_w, tok_w)],
            out_hbm.at[pl.ds(wid * tok_w, tok_w),
                       pl.ds(g * var_dim, var_dim)])


def _sc_gather(table, idx0, idx1, n_tok):
    var_dim = table.shape[-1]
    info = plsc.get_sparse_core_info()
    nw = info.num_cores * info.num_subcores
    tok_w = n_tok // nw
    mesh = plsc.VectorSubcoreMesh(core_axis_name="c", subcore_axis_name="s")
    body = functools.partial(_gather_body, num_cores=info.num_cores,
                             tok_w=tok_w, var_dim=var_dim)
    f = pl.kernel(
        body,
        out_type=jax.ShapeDtypeStruct((n_tok, _G * var_dim), jnp.float32),
        mesh=mesh,
        scratch_types=[
            pltpu.VMEM((_G * tok_w // _CHUNK, _CHUNK), jnp.int32),
            pltpu.VMEM((_G * tok_w, var_dim), jnp.float32),
            pltpu.SemaphoreType.DMA,
        ],
    )
    return f(table, idx0, idx1)


def kernel(x, codebook, W, b):
    bsz, tsz, fsz = x.shape
    rows = bsz * tsz
    gv = W.shape[0]
    num_vars = gv // _G
    var_dim = codebook.shape[-1]

    lt, idx0, idx1 = _project_and_select(
        x.reshape(rows, fsz), W, b, num_vars)
    q2d = _sc_gather(codebook.reshape(gv, var_dim), idx0, idx1, rows)
    cperp, pperp = _perplexities(lt, idx0, idx1, num_vars)
    q = q2d.reshape(bsz, tsz, _G * var_dim)
    return (q, gv, cperp.reshape(()), pperp.reshape(()))
